# baseline (speedup 1.0000x reference)
import functools

import jax
import jax.numpy as jnp
from jax import lax
from jax.experimental import pallas as pl
from jax.experimental.pallas import tpu as pltpu
from jax.experimental.pallas import tpu_sc as plsc

SEQ = 64
BATCH = 32
EMB = 1024
HID = 1024
VOCAB = 10000
G4 = 4 * HID
TOK = SEQ * BATCH


def _sc_gather(table, idx):
    B = idx.shape[0]
    D = table.shape[1]
    info = plsc.get_sparse_core_info()
    nw = info.num_cores * info.num_subcores
    b_per_w = B // nw
    mesh = plsc.VectorSubcoreMesh(core_axis_name="c", subcore_axis_name="s")

    @functools.partial(
        pl.kernel,
        mesh=mesh,
        out_type=jax.ShapeDtypeStruct((B, D), jnp.float32),
        scratch_types=[
            pltpu.VMEM((b_per_w,), jnp.int32),
            pltpu.VMEM((b_per_w, D), jnp.float32),
            pltpu.SemaphoreType.DMA,
        ],
    )
    def gk(table_hbm, idx_hbm, out_hbm, idx_v, rows_v, sem):
        wid = lax.axis_index("s") * info.num_cores + lax.axis_index("c")
        base = wid * b_per_w
        pltpu.sync_copy(idx_hbm.at[pl.ds(base, b_per_w)], idx_v)
        pltpu.async_copy(table_hbm.at[idx_v], rows_v, sem).wait()
        pltpu.sync_copy(rows_v, out_hbm.at[pl.ds(base, b_per_w)])

    return gk(table, idx)


def _matmul_bias(a, w, b, n_block, out_dtype=jnp.float32):
    M, K = a.shape
    N = w.shape[0]
    nb = pl.cdiv(N, n_block)

    def mk(a_ref, w_ref, b_ref, o_ref):
        acc = (
            lax.dot_general(
                a_ref[...].astype(jnp.bfloat16),
                w_ref[...].astype(jnp.bfloat16),
                (((1,), (1,)), ((), ())),
                preferred_element_type=jnp.float32,
            )
            + b_ref[...]
        )
        o_ref[...] = acc.astype(out_dtype)

    return pl.pallas_call(
        mk,
        grid=(nb,),
        in_specs=[
            pl.BlockSpec((M, K), lambda n: (0, 0)),
            pl.BlockSpec((n_block, K), lambda n: (n, 0)),
            pl.BlockSpec((1, n_block), lambda n: (0, n)),
        ],
        out_specs=pl.BlockSpec((M, n_block), lambda n: (0, n)),
        out_shape=jax.ShapeDtypeStruct((M, N), out_dtype),
    )(a, w, b)


def _lstm_scan(xg, wh, h0, c0):
    U = 4
    NG = SEQ // U

    def body(x_ref, w_ref, h0_ref, c0_ref, out_ref, hT_ref, cT_ref, h_s, c_s,
             wt_s):
        gidx = pl.program_id(0)

        @pl.when(gidx == 0)
        def _():
            h_s[...] = h0_ref[...]
            c_s[...] = c0_ref[...]
            wt_s[...] = w_ref[...].T.astype(jnp.bfloat16)

        h = h_s[...]
        c = c_s[...]
        for u in range(U):
            gates = x_ref[u].astype(jnp.float32) + jnp.dot(
                h.astype(jnp.bfloat16), wt_s[...],
                preferred_element_type=jnp.float32,
            )
            i = jax.nn.sigmoid(gates[:, 0:HID])
            f = jax.nn.sigmoid(gates[:, HID : 2 * HID])
            g = jnp.tanh(gates[:, 2 * HID : 3 * HID])
            o = jax.nn.sigmoid(gates[:, 3 * HID : 4 * HID])
            c = f * c + i * g
            h = o * jnp.tanh(c)
            out_ref[u] = h.astype(jnp.bfloat16)
        h_s[...] = h
        c_s[...] = c

        @pl.when(gidx == NG - 1)
        def _():
            hT_ref[...] = h
            cT_ref[...] = c

    return pl.pallas_call(
        body,
        grid=(NG,),
        in_specs=[
            pl.BlockSpec((U, BATCH, G4), lambda t: (t, 0, 0)),
            pl.BlockSpec((G4, HID), lambda t: (0, 0)),
            pl.BlockSpec((BATCH, HID), lambda t: (0, 0)),
            pl.BlockSpec((BATCH, HID), lambda t: (0, 0)),
        ],
        out_specs=[
            pl.BlockSpec((U, BATCH, HID), lambda t: (t, 0, 0)),
            pl.BlockSpec((BATCH, HID), lambda t: (0, 0)),
            pl.BlockSpec((BATCH, HID), lambda t: (0, 0)),
        ],
        out_shape=[
            jax.ShapeDtypeStruct((SEQ, BATCH, HID), jnp.bfloat16),
            jax.ShapeDtypeStruct((BATCH, HID), jnp.float32),
            jax.ShapeDtypeStruct((BATCH, HID), jnp.float32),
        ],
        scratch_shapes=[
            pltpu.VMEM((BATCH, HID), jnp.float32),
            pltpu.VMEM((BATCH, HID), jnp.float32),
            pltpu.VMEM((HID, G4), jnp.bfloat16),
        ],
    )(xg, wh, h0, c0)


def kernel(x, h0, c0, emb, W_ih0, W_hh0, b_ih0, b_hh0, W_ih1, W_hh1, b_ih1,
           b_hh1, fc_w, fc_b):
    idx = x.reshape(-1).astype(jnp.int32)
    e = _sc_gather(emb, idx)

    b0 = (b_ih0 + b_hh0).reshape(1, G4)
    b1 = (b_ih1 + b_hh1).reshape(1, G4)

    x0 = _matmul_bias(e, W_ih0, b0, 1024, jnp.bfloat16)
    out0, h0f, c0f = _lstm_scan(x0.reshape(SEQ, BATCH, G4), W_hh0, h0[0], c0[0])

    x1 = _matmul_bias(out0.reshape(TOK, HID), W_ih1, b1, 1024, jnp.bfloat16)
    out1, h1f, c1f = _lstm_scan(x1.reshape(SEQ, BATCH, G4), W_hh1, h0[1], c0[1])

    logits = _matmul_bias(out1.reshape(TOK, HID), fc_w, fc_b.reshape(1, VOCAB), 1024)

    return (
        logits.reshape(SEQ, BATCH, VOCAB),
        jnp.stack([h0f, h1f], axis=0),
        jnp.stack([c0f, c1f], axis=0),
    )


# speedup vs baseline: 1.0130x; 1.0130x over previous
import functools

import jax
import jax.numpy as jnp
from jax import lax
from jax.experimental import pallas as pl
from jax.experimental.pallas import tpu as pltpu
from jax.experimental.pallas import tpu_sc as plsc

SEQ = 64
BATCH = 32
EMB = 1024
HID = 1024
VOCAB = 10000
G4 = 4 * HID
TOK = SEQ * BATCH
U = 4
NG = SEQ // U


def _sc_gather(table, idx):
    B = idx.shape[0]
    D = table.shape[1]
    info = plsc.get_sparse_core_info()
    nw = info.num_cores * info.num_subcores
    b_per_w = B // nw
    mesh = plsc.VectorSubcoreMesh(core_axis_name="c", subcore_axis_name="s")

    @functools.partial(
        pl.kernel,
        mesh=mesh,
        out_type=jax.ShapeDtypeStruct((B, D), jnp.float32),
        scratch_types=[
            pltpu.VMEM((b_per_w,), jnp.int32),
            pltpu.VMEM((b_per_w, D), jnp.float32),
            pltpu.SemaphoreType.DMA,
        ],
    )
    def gk(table_hbm, idx_hbm, out_hbm, idx_v, rows_v, sem):
        wid = lax.axis_index("s") * info.num_cores + lax.axis_index("c")
        base = wid * b_per_w
        pltpu.sync_copy(idx_hbm.at[pl.ds(base, b_per_w)], idx_v)
        pltpu.async_copy(table_hbm.at[idx_v], rows_v, sem).wait()
        pltpu.sync_copy(rows_v, out_hbm.at[pl.ds(base, b_per_w)])

    return gk(table, idx)


def _matmul_bias(a, w, b, n_block, out_dtype=jnp.float32):
    M, K = a.shape
    N = w.shape[0]
    nb = pl.cdiv(N, n_block)

    def mk(a_ref, w_ref, b_ref, o_ref):
        acc = (
            lax.dot_general(
                a_ref[...].astype(jnp.bfloat16),
                w_ref[...].astype(jnp.bfloat16),
                (((1,), (1,)), ((), ())),
                preferred_element_type=jnp.float32,
            )
            + b_ref[...]
        )
        o_ref[...] = acc.astype(out_dtype)

    return pl.pallas_call(
        mk,
        grid=(nb,),
        in_specs=[
            pl.BlockSpec((M, K), lambda n: (0, 0)),
            pl.BlockSpec((n_block, K), lambda n: (n, 0)),
            pl.BlockSpec((1, n_block), lambda n: (0, n)),
        ],
        out_specs=pl.BlockSpec((M, n_block), lambda n: (0, n)),
        out_shape=jax.ShapeDtypeStruct((M, N), out_dtype),
    )(a, w, b)


def _lstm_then_matmul(xg, wh, h0, c0, w2, b2, n_block, out_dtype):
    N2 = w2.shape[0]
    nb2 = pl.cdiv(N2, n_block)

    def body(x_ref, wh_ref, h0_ref, c0_ref, w2_ref, b2_ref,
             y_ref, hT_ref, cT_ref, h_s, c_s, wt_s, out_s):
        g = pl.program_id(0)

        @pl.when(g == 0)
        def _():
            h_s[...] = h0_ref[...]
            c_s[...] = c0_ref[...]
            wt_s[...] = wh_ref[...].T

        @pl.when(g < NG)
        def _():
            h = h_s[...]
            c = c_s[...]
            for u in range(U):
                gates = x_ref[u].astype(jnp.float32) + jnp.dot(
                    h, wt_s[...], preferred_element_type=jnp.float32
                )
                i = jax.nn.sigmoid(gates[:, 0:HID])
                f = jax.nn.sigmoid(gates[:, HID : 2 * HID])
                gg = jnp.tanh(gates[:, 2 * HID : 3 * HID])
                o = jax.nn.sigmoid(gates[:, 3 * HID : 4 * HID])
                c = f * c + i * gg
                h = o * jnp.tanh(c)
                out_s[pl.ds((g * U + u) * BATCH, BATCH), :] = h.astype(
                    jnp.bfloat16
                )
            h_s[...] = h
            c_s[...] = c

            @pl.when(g == NG - 1)
            def _():
                hT_ref[...] = h
                cT_ref[...] = c

        @pl.when(g >= NG)
        def _():
            acc = (
                lax.dot_general(
                    out_s[...],
                    w2_ref[...].astype(jnp.bfloat16),
                    (((1,), (1,)), ((), ())),
                    preferred_element_type=jnp.float32,
                )
                + b2_ref[...]
            )
            y_ref[...] = acc.astype(out_dtype)

    return pl.pallas_call(
        body,
        grid=(NG + nb2,),
        in_specs=[
            pl.BlockSpec((U, BATCH, G4), lambda g: (jnp.minimum(g, NG - 1), 0, 0)),
            pl.BlockSpec((G4, HID), lambda g: (0, 0)),
            pl.BlockSpec((BATCH, HID), lambda g: (0, 0)),
            pl.BlockSpec((BATCH, HID), lambda g: (0, 0)),
            pl.BlockSpec((n_block, HID), lambda g: (jnp.maximum(g - NG, 0), 0)),
            pl.BlockSpec((1, n_block), lambda g: (0, jnp.maximum(g - NG, 0))),
        ],
        out_specs=[
            pl.BlockSpec((TOK, n_block), lambda g: (0, jnp.maximum(g - NG, 0))),
            pl.BlockSpec((BATCH, HID), lambda g: (0, 0)),
            pl.BlockSpec((BATCH, HID), lambda g: (0, 0)),
        ],
        out_shape=[
            jax.ShapeDtypeStruct((TOK, N2), out_dtype),
            jax.ShapeDtypeStruct((BATCH, HID), jnp.float32),
            jax.ShapeDtypeStruct((BATCH, HID), jnp.float32),
        ],
        scratch_shapes=[
            pltpu.VMEM((BATCH, HID), jnp.float32),
            pltpu.VMEM((BATCH, HID), jnp.float32),
            pltpu.VMEM((HID, G4), jnp.float32),
            pltpu.VMEM((TOK, HID), jnp.bfloat16),
        ],
    )(xg, wh, h0, c0, w2, b2)


def kernel(x, h0, c0, emb, W_ih0, W_hh0, b_ih0, b_hh0, W_ih1, W_hh1, b_ih1,
           b_hh1, fc_w, fc_b):
    idx = x.reshape(-1).astype(jnp.int32)
    e = _sc_gather(emb, idx)

    b0 = (b_ih0 + b_hh0).reshape(1, G4)
    b1 = (b_ih1 + b_hh1).reshape(1, G4)

    x0 = _matmul_bias(e, W_ih0, b0, 1024, jnp.bfloat16)
    x1, h0f, c0f = _lstm_then_matmul(
        x0.reshape(SEQ, BATCH, G4), W_hh0, h0[0], c0[0],
        W_ih1, b1, 512, jnp.bfloat16,
    )
    logits, h1f, c1f = _lstm_then_matmul(
        x1.reshape(SEQ, BATCH, G4), W_hh1, h0[1], c0[1],
        fc_w, fc_b.reshape(1, VOCAB), 512, jnp.float32,
    )

    return (
        logits.reshape(SEQ, BATCH, VOCAB),
        jnp.stack([h0f, h1f], axis=0),
        jnp.stack([c0f, c1f], axis=0),
    )


# speedup vs baseline: 1.0239x; 1.0107x over previous
import functools

import jax
import jax.numpy as jnp
from jax import lax
from jax.experimental import pallas as pl
from jax.experimental.pallas import tpu as pltpu
from jax.experimental.pallas import tpu_sc as plsc

SEQ = 64
BATCH = 32
EMB = 1024
HID = 1024
VOCAB = 10000
G4 = 4 * HID
TOK = SEQ * BATCH
U = 8
NG = SEQ // U


def _sc_gather(table, idx):
    B = idx.shape[0]
    D = table.shape[1]
    info = plsc.get_sparse_core_info()
    nw = info.num_cores * info.num_subcores
    b_per_w = B // nw
    mesh = plsc.VectorSubcoreMesh(core_axis_name="c", subcore_axis_name="s")

    @functools.partial(
        pl.kernel,
        mesh=mesh,
        out_type=jax.ShapeDtypeStruct((B, D), jnp.float32),
        scratch_types=[
            pltpu.VMEM((b_per_w,), jnp.int32),
            pltpu.VMEM((b_per_w, D), jnp.float32),
            pltpu.SemaphoreType.DMA,
        ],
    )
    def gk(table_hbm, idx_hbm, out_hbm, idx_v, rows_v, sem):
        wid = lax.axis_index("s") * info.num_cores + lax.axis_index("c")
        base = wid * b_per_w
        pltpu.sync_copy(idx_hbm.at[pl.ds(base, b_per_w)], idx_v)
        pltpu.async_copy(table_hbm.at[idx_v], rows_v, sem).wait()
        pltpu.sync_copy(rows_v, out_hbm.at[pl.ds(base, b_per_w)])

    return gk(table, idx)


def _matmul_bias(a, w, b, n_block, out_dtype=jnp.float32):
    M, K = a.shape
    N = w.shape[0]
    nb = pl.cdiv(N, n_block)

    def mk(a_ref, w_ref, b_ref, o_ref):
        acc = (
            lax.dot_general(
                a_ref[...].astype(jnp.bfloat16),
                w_ref[...].astype(jnp.bfloat16),
                (((1,), (1,)), ((), ())),
                preferred_element_type=jnp.float32,
            )
            + b_ref[...]
        )
        o_ref[...] = acc.astype(out_dtype)

    return pl.pallas_call(
        mk,
        grid=(nb,),
        in_specs=[
            pl.BlockSpec((M, K), lambda n: (0, 0)),
            pl.BlockSpec((n_block, K), lambda n: (n, 0)),
            pl.BlockSpec((1, n_block), lambda n: (0, n)),
        ],
        out_specs=pl.BlockSpec((M, n_block), lambda n: (0, n)),
        out_shape=jax.ShapeDtypeStruct((M, N), out_dtype),
    )(a, w, b)


def _lstm_then_matmul(xg, wh, h0, c0, w2, b2, n_block, out_dtype):
    N2 = w2.shape[0]
    nb2 = pl.cdiv(N2, n_block)

    def body(x_ref, wh_ref, h0_ref, c0_ref, w2_ref, b2_ref,
             y_ref, hT_ref, cT_ref, h_s, c_s, out_s):
        g = pl.program_id(0)

        @pl.when(g == 0)
        def _():
            h_s[...] = h0_ref[...]
            c_s[...] = c0_ref[...]

        @pl.when(g < NG)
        def _():
            h = h_s[...]
            c = c_s[...]
            for u in range(U):
                gates = x_ref[u].astype(jnp.float32) + jnp.dot(
                    h, wh_ref[...], preferred_element_type=jnp.float32
                )
                i = jax.nn.sigmoid(gates[:, 0:HID])
                f = jax.nn.sigmoid(gates[:, HID : 2 * HID])
                gg = jnp.tanh(gates[:, 2 * HID : 3 * HID])
                o = jax.nn.sigmoid(gates[:, 3 * HID : 4 * HID])
                c = f * c + i * gg
                h = o * jnp.tanh(c)
                out_s[pl.ds((g * U + u) * BATCH, BATCH), :] = h.astype(
                    jnp.bfloat16
                )
            h_s[...] = h
            c_s[...] = c

            @pl.when(g == NG - 1)
            def _():
                hT_ref[...] = h
                cT_ref[...] = c

        @pl.when(g >= NG)
        def _():
            acc = (
                lax.dot_general(
                    out_s[...],
                    w2_ref[...].astype(jnp.bfloat16),
                    (((1,), (1,)), ((), ())),
                    preferred_element_type=jnp.float32,
                )
                + b2_ref[...]
            )
            y_ref[...] = acc.astype(out_dtype)

    return pl.pallas_call(
        body,
        grid=(NG + nb2,),
        in_specs=[
            pl.BlockSpec((U, BATCH, G4), lambda g: (jnp.minimum(g, NG - 1), 0, 0)),
            pl.BlockSpec((HID, G4), lambda g: (0, 0)),
            pl.BlockSpec((BATCH, HID), lambda g: (0, 0)),
            pl.BlockSpec((BATCH, HID), lambda g: (0, 0)),
            pl.BlockSpec((n_block, HID), lambda g: (jnp.maximum(g - NG, 0), 0)),
            pl.BlockSpec((1, n_block), lambda g: (0, jnp.maximum(g - NG, 0))),
        ],
        out_specs=[
            pl.BlockSpec((TOK, n_block), lambda g: (0, jnp.maximum(g - NG, 0))),
            pl.BlockSpec((BATCH, HID), lambda g: (0, 0)),
            pl.BlockSpec((BATCH, HID), lambda g: (0, 0)),
        ],
        out_shape=[
            jax.ShapeDtypeStruct((TOK, N2), out_dtype),
            jax.ShapeDtypeStruct((BATCH, HID), jnp.float32),
            jax.ShapeDtypeStruct((BATCH, HID), jnp.float32),
        ],
        scratch_shapes=[
            pltpu.VMEM((BATCH, HID), jnp.float32),
            pltpu.VMEM((BATCH, HID), jnp.float32),
            pltpu.VMEM((TOK, HID), jnp.bfloat16),
        ],
    )(xg, wh, h0, c0, w2, b2)


def kernel(x, h0, c0, emb, W_ih0, W_hh0, b_ih0, b_hh0, W_ih1, W_hh1, b_ih1,
           b_hh1, fc_w, fc_b):
    idx = x.reshape(-1).astype(jnp.int32)
    e = _sc_gather(emb, idx)

    b0 = (b_ih0 + b_hh0).reshape(1, G4)
    b1 = (b_ih1 + b_hh1).reshape(1, G4)

    x0 = _matmul_bias(e, W_ih0, b0, 1024, jnp.bfloat16)
    x1, h0f, c0f = _lstm_then_matmul(
        x0.reshape(SEQ, BATCH, G4), W_hh0.T, h0[0], c0[0],
        W_ih1, b1, 1024, jnp.bfloat16,
    )
    logits, h1f, c1f = _lstm_then_matmul(
        x1.reshape(SEQ, BATCH, G4), W_hh1.T, h0[1], c0[1],
        fc_w, fc_b.reshape(1, VOCAB), 1024, jnp.float32,
    )

    return (
        logits.reshape(SEQ, BATCH, VOCAB),
        jnp.stack([h0f, h1f], axis=0),
        jnp.stack([c0f, c1f], axis=0),
    )
